# baseline (speedup 1.0000x reference)
import functools

import jax
import jax.numpy as jnp
from jax import lax
from jax.experimental import pallas as pl
from jax.experimental.pallas import tpu as pltpu
from jax.experimental.pallas import tpu_sc as plsc

NC = 2
NS = 16
L = 16
C = 16


@functools.partial(jax.jit, static_argnames=("batch", "seq_len", "d"))
def _sc_fused_lookup(idx_s, idx_r, idx_d, pe, st, rt, dt, wvec, *, batch, seq_len, d):
    NW = NC * NS
    s_per_w = seq_len // NW
    n_sc = s_per_w // C
    n_stage = n_sc * batch
    dw = d // 2
    chunks = d // (2 * L)

    mesh = plsc.VectorSubcoreMesh(
        core_axis_name="c", subcore_axis_name="s",
        num_cores=NC, num_subcores=NS)

    @functools.partial(
        pl.kernel,
        out_type=jax.ShapeDtypeStruct((batch * seq_len, d), jnp.float32),
        mesh=mesh,
        compiler_params=pltpu.CompilerParams(needs_layout_passes=False),
        scratch_types=[
            pltpu.VMEM((batch, n_sc, C), jnp.int32),
            pltpu.VMEM((batch, n_sc, C), jnp.int32),
            pltpu.VMEM((batch, n_sc, C), jnp.int32),
            pltpu.VMEM((C, dw), jnp.int32),
            pltpu.VMEM((2, C, dw), jnp.int32),
            pltpu.VMEM((2, C, dw), jnp.int32),
            pltpu.VMEM((2, C, dw), jnp.int32),
            pltpu.VMEM((2, C, d), jnp.float32),
            pltpu.VMEM((4, L), jnp.int32),
            pltpu.SemaphoreType.DMA,
            pltpu.SemaphoreType.DMA,
            pltpu.SemaphoreType.DMA,
            pltpu.SemaphoreType.DMA,
            pltpu.SemaphoreType.DMA,
        ],
    )
    def body(sc_hbm, ro_hbm, di_hbm, pe_hbm, st_hbm, rt_hbm, dt_hbm, w_hbm,
             out_hbm, idx_sv, idx_rv, idx_dv, pe_v, g1, g2, g3, ob, w_v,
             sem_g0, sem_g1, sem_o0, sem_o1, sem_pe):
        wid = lax.axis_index("s") * NC + lax.axis_index("c")
        s_base = wid * s_per_w

        pltpu.sync_copy(sc_hbm.at[wid], idx_sv)
        pltpu.sync_copy(ro_hbm.at[wid], idx_rv)
        pltpu.sync_copy(di_hbm.at[wid], idx_dv)
        pltpu.sync_copy(w_hbm, w_v)
        w0 = plsc.bitcast(w_v[0, :], jnp.bfloat16)
        w1 = plsc.bitcast(w_v[1, :], jnp.bfloat16)
        w2 = plsc.bitcast(w_v[2, :], jnp.bfloat16)
        w3 = plsc.bitcast(w_v[3, :], jnp.bfloat16)

        sem_g = (sem_g0, sem_g1)
        sem_o = (sem_o0, sem_o1)
        gbufs = ((g1.at[0], g2.at[0], g3.at[0]), (g1.at[1], g2.at[1], g3.at[1]))
        obufs = (ob.at[0], ob.at[1])

        def stage_tb(ls):
            return ls // batch, lax.rem(ls, batch)

        def issue_gathers(ls, p):
            t, b = stage_tb(ls)
            b1, b2, b3 = gbufs[p]
            pltpu.async_copy(st_hbm.at[idx_sv.at[b, t]], b1, sem_g[p])
            pltpu.async_copy(rt_hbm.at[idx_rv.at[b, t]], b2, sem_g[p])
            pltpu.async_copy(dt_hbm.at[idx_dv.at[b, t]], b3, sem_g[p])

        def wait_gathers(ls, p):
            t, b = stage_tb(ls)
            b1, b2, b3 = gbufs[p]
            pltpu.make_async_copy(st_hbm.at[idx_sv.at[b, t]], b1, sem_g[p]).wait()
            pltpu.make_async_copy(rt_hbm.at[idx_rv.at[b, t]], b2, sem_g[p]).wait()
            pltpu.make_async_copy(dt_hbm.at[idx_dv.at[b, t]], b3, sem_g[p]).wait()

        def out_rows(ls):
            t, b = stage_tb(ls)
            return b * seq_len + s_base + t * C

        def issue_out(ls, p):
            pltpu.async_copy(obufs[p], out_hbm.at[pl.ds(out_rows(ls), C)],
                             sem_o[p])

        def wait_out(ls, p):
            pltpu.make_async_copy(obufs[p],
                                  out_hbm.at[pl.ds(out_rows(ls), C)],
                                  sem_o[p]).wait()

        def issue_pe(t):
            pltpu.async_copy(pe_hbm.at[pl.ds(s_base + t * C, C)], pe_v, sem_pe)

        def wait_pe(t):
            pltpu.make_async_copy(pe_hbm.at[pl.ds(s_base + t * C, C)], pe_v,
                                  sem_pe).wait()

        def compute(p):
            b1, b2, b3 = gbufs[p]
            obuf = obufs[p]

            def row(i, carry2):
                def pair(jj, carry3):
                    for u in range(2):
                        j = jj * 2 + u
                        sl = pl.ds(j * L, L)

                        def as_bf(ref):
                            return plsc.bitcast(ref[i, sl], jnp.bfloat16)

                        acc = (as_bf(pe_v) * w0 + as_bf(b1) * w1
                               + as_bf(b2) * w2 + as_bf(b3) * w3)
                        lo, hi = plsc.unpack(
                            acc, format=plsc.PackFormat.INTERLEAVED,
                            preferred_element_type=jnp.float32)
                        obuf[i, pl.ds(j * 2 * L, L)] = lo
                        obuf[i, pl.ds((j * 2 + 1) * L, L)] = hi
                    return carry3
                return lax.fori_loop(0, chunks // 2, pair, carry2)

            lax.fori_loop(0, C, row, 0)

        issue_pe(0)
        issue_gathers(0, 0)
        issue_gathers(1, 1)

        def iteration(k, carry):
            ls0 = 2 * k

            wait_gathers(ls0, 0)

            @pl.when(k > 0)
            def _():
                wait_out(ls0 - 2, 0)

            @pl.when(lax.rem(ls0, batch) == 0)
            def _():
                wait_pe(ls0 // batch)
            compute(0)

            @pl.when((lax.rem(ls0, batch) == batch - 1)
                     & (ls0 // batch + 1 < n_sc))
            def _():
                issue_pe(ls0 // batch + 1)
            issue_out(ls0, 0)

            @pl.when(k < n_stage // 2 - 1)
            def _():
                issue_gathers(ls0 + 2, 0)

            ls1 = ls0 + 1

            wait_gathers(ls1, 1)

            @pl.when(k > 0)
            def _():
                wait_out(ls1 - 2, 1)

            @pl.when(lax.rem(ls1, batch) == 0)
            def _():
                wait_pe(ls1 // batch)
            compute(1)

            @pl.when((lax.rem(ls1, batch) == batch - 1)
                     & (ls1 // batch + 1 < n_sc))
            def _():
                issue_pe(ls1 // batch + 1)
            issue_out(ls1, 1)

            @pl.when(k < n_stage // 2 - 1)
            def _():
                issue_gathers(ls1 + 2, 1)
            return carry

        lax.fori_loop(0, n_stage // 2, iteration, 0)
        wait_out(n_stage - 2, 0)
        wait_out(n_stage - 1, 1)

    return body(idx_s, idx_r, idx_d, pe, st, rt, dt, wvec)


def kernel(positions, scales, rotations, distances, pe, scale_table,
           rotation_table, distance_table, fusion_weights):
    b, s = positions.shape
    d = pe.shape[1]
    NW = NC * NS
    n_sc = s // NW // C
    w = jax.nn.softmax(fusion_weights.astype(jnp.float32), axis=0)
    wb = w.astype(jnp.bfloat16)
    wpair = jnp.stack([wb, wb], axis=-1)
    wword = jax.lax.bitcast_convert_type(wpair, jnp.int32)
    wvec = jnp.broadcast_to(wword[:, None], (4, L))
    shape = (b, NW, n_sc, C)
    idx_s = scales.reshape(shape).astype(jnp.int32).transpose(1, 0, 2, 3)
    idx_r = rotations.reshape(shape).astype(jnp.int32).transpose(1, 0, 2, 3)
    idx_d = distances.reshape(shape).astype(jnp.int32).transpose(1, 0, 2, 3)

    def pack_bf16_rows(tbl):
        v = tbl.shape[0]
        xb = tbl.astype(jnp.bfloat16).reshape(v, d // (2 * L), 2, L)
        xb = xb.swapaxes(2, 3)
        return jax.lax.bitcast_convert_type(xb, jnp.int32).reshape(v, d // 2)

    out = _sc_fused_lookup(idx_s, idx_r, idx_d, pack_bf16_rows(pe),
                           pack_bf16_rows(scale_table),
                           pack_bf16_rows(rotation_table),
                           pack_bf16_rows(distance_table), wvec,
                           batch=b, seq_len=s, d=d)
    return out.reshape(b, s, d)


# speedup vs baseline: 1.2053x; 1.2053x over previous
import functools

import jax
import jax.numpy as jnp
from jax import lax
from jax.experimental import pallas as pl
from jax.experimental.pallas import tpu as pltpu
from jax.experimental.pallas import tpu_sc as plsc

NC = 2
NS = 16
L = 16
C = 16


@functools.partial(jax.jit, static_argnames=("batch", "seq_len", "d"))
def _sc_fused_lookup(idx_s, idx_r, idx_d, pe, st, rt, dt, wvec, *, batch, seq_len, d):
    NW = NC * NS
    s_per_w = seq_len // NW
    n_sc = s_per_w // C
    n_stage = n_sc * batch
    dw = d // 2
    chunks = d // (2 * L)

    mesh = plsc.VectorSubcoreMesh(
        core_axis_name="c", subcore_axis_name="s",
        num_cores=NC, num_subcores=NS)

    @functools.partial(
        pl.kernel,
        out_type=jax.ShapeDtypeStruct((batch * seq_len, d), jnp.float32),
        mesh=mesh,
        scratch_types=[
            pltpu.VMEM((batch, n_sc, C), jnp.int32),
            pltpu.VMEM((batch, n_sc, C), jnp.int32),
            pltpu.VMEM((batch, n_sc, C), jnp.int32),
            pltpu.VMEM((C, d), jnp.float32),
            pltpu.VMEM((2, C, dw), jnp.int32),
            pltpu.VMEM((2, C, dw), jnp.int32),
            pltpu.VMEM((2, C, dw), jnp.int32),
            pltpu.VMEM((2, C, d), jnp.float32),
            pltpu.VMEM((4, L), jnp.float32),
            pltpu.SemaphoreType.DMA,
            pltpu.SemaphoreType.DMA,
            pltpu.SemaphoreType.DMA,
            pltpu.SemaphoreType.DMA,
            pltpu.SemaphoreType.DMA,
        ],
    )
    def body(sc_hbm, ro_hbm, di_hbm, pe_hbm, st_hbm, rt_hbm, dt_hbm, w_hbm,
             out_hbm, idx_sv, idx_rv, idx_dv, pe_v, g1, g2, g3, ob, w_v,
             sem_g0, sem_g1, sem_o0, sem_o1, sem_pe):
        wid = lax.axis_index("s") * NC + lax.axis_index("c")
        s_base = wid * s_per_w

        pltpu.sync_copy(sc_hbm.at[wid], idx_sv)
        pltpu.sync_copy(ro_hbm.at[wid], idx_rv)
        pltpu.sync_copy(di_hbm.at[wid], idx_dv)
        pltpu.sync_copy(w_hbm, w_v)
        w0 = w_v[0, :]
        w1 = w_v[1, :]
        w2 = w_v[2, :]
        w3 = w_v[3, :]

        sem_g = (sem_g0, sem_g1)
        sem_o = (sem_o0, sem_o1)
        gbufs = ((g1.at[0], g2.at[0], g3.at[0]), (g1.at[1], g2.at[1], g3.at[1]))
        obufs = (ob.at[0], ob.at[1])

        def stage_tb(ls):
            return ls // batch, lax.rem(ls, batch)

        def issue_gathers(ls, p):
            t, b = stage_tb(ls)
            b1, b2, b3 = gbufs[p]
            pltpu.async_copy(st_hbm.at[idx_sv.at[b, t]], b1, sem_g[p])
            pltpu.async_copy(rt_hbm.at[idx_rv.at[b, t]], b2, sem_g[p])
            pltpu.async_copy(dt_hbm.at[idx_dv.at[b, t]], b3, sem_g[p])

        def wait_gathers(ls, p):
            t, b = stage_tb(ls)
            b1, b2, b3 = gbufs[p]
            pltpu.make_async_copy(st_hbm.at[idx_sv.at[b, t]], b1, sem_g[p]).wait()
            pltpu.make_async_copy(rt_hbm.at[idx_rv.at[b, t]], b2, sem_g[p]).wait()
            pltpu.make_async_copy(dt_hbm.at[idx_dv.at[b, t]], b3, sem_g[p]).wait()

        def out_rows(ls):
            t, b = stage_tb(ls)
            return b * seq_len + s_base + t * C

        def issue_out(ls, p):
            pltpu.async_copy(obufs[p], out_hbm.at[pl.ds(out_rows(ls), C)],
                             sem_o[p])

        def wait_out(ls, p):
            pltpu.make_async_copy(obufs[p],
                                  out_hbm.at[pl.ds(out_rows(ls), C)],
                                  sem_o[p]).wait()

        def issue_pe(t):
            pltpu.async_copy(pe_hbm.at[pl.ds(s_base + t * C, C)], pe_v, sem_pe)

        def wait_pe(t):
            pltpu.make_async_copy(pe_hbm.at[pl.ds(s_base + t * C, C)], pe_v,
                                  sem_pe).wait()

        def compute(p):
            b1, b2, b3 = gbufs[p]
            obuf = obufs[p]

            def row(i, carry2):
                def pair(jj, carry3):
                    for u in range(2):
                        j = jj * 2 + u
                        sl = pl.ds(j * L, L)

                        def unp(ref):
                            word = ref[i, sl]
                            lo_v = jax.lax.bitcast_convert_type(
                                word << 16, jnp.float32)
                            hi_v = jax.lax.bitcast_convert_type(
                                word, jnp.float32)
                            return lo_v, hi_v

                        a_lo, a_hi = unp(b1)
                        c_lo, c_hi = unp(b2)
                        e_lo, e_hi = unp(b3)
                        p_lo = pe_v[i, pl.ds(j * 2 * L, L)]
                        p_hi = pe_v[i, pl.ds((j * 2 + 1) * L, L)]
                        lo = p_lo * w0 + a_lo * w1 + c_lo * w2 + e_lo * w3
                        hi = p_hi * w0 + a_hi * w1 + c_hi * w2 + e_hi * w3
                        obuf[i, pl.ds(j * 2 * L, L)] = lo
                        obuf[i, pl.ds((j * 2 + 1) * L, L)] = hi
                    return carry3
                return lax.fori_loop(0, chunks // 2, pair, carry2)

            lax.fori_loop(0, C, row, 0)

        issue_pe(0)
        issue_gathers(0, 0)
        issue_gathers(1, 1)

        def iteration(k, carry):
            ls0 = 2 * k

            wait_gathers(ls0, 0)

            @pl.when(k > 0)
            def _():
                wait_out(ls0 - 2, 0)

            @pl.when(lax.rem(ls0, batch) == 0)
            def _():
                wait_pe(ls0 // batch)
            compute(0)

            @pl.when((lax.rem(ls0, batch) == batch - 1)
                     & (ls0 // batch + 1 < n_sc))
            def _():
                issue_pe(ls0 // batch + 1)
            issue_out(ls0, 0)

            @pl.when(k < n_stage // 2 - 1)
            def _():
                issue_gathers(ls0 + 2, 0)

            ls1 = ls0 + 1

            wait_gathers(ls1, 1)

            @pl.when(k > 0)
            def _():
                wait_out(ls1 - 2, 1)

            @pl.when(lax.rem(ls1, batch) == 0)
            def _():
                wait_pe(ls1 // batch)
            compute(1)

            @pl.when((lax.rem(ls1, batch) == batch - 1)
                     & (ls1 // batch + 1 < n_sc))
            def _():
                issue_pe(ls1 // batch + 1)
            issue_out(ls1, 1)

            @pl.when(k < n_stage // 2 - 1)
            def _():
                issue_gathers(ls1 + 2, 1)
            return carry

        lax.fori_loop(0, n_stage // 2, iteration, 0)
        wait_out(n_stage - 2, 0)
        wait_out(n_stage - 1, 1)

    return body(idx_s, idx_r, idx_d, pe, st, rt, dt, wvec)


def kernel(positions, scales, rotations, distances, pe, scale_table,
           rotation_table, distance_table, fusion_weights):
    b, s = positions.shape
    d = pe.shape[1]
    NW = NC * NS
    n_sc = s // NW // C
    w = jax.nn.softmax(fusion_weights.astype(jnp.float32), axis=0)
    wvec = jnp.broadcast_to(w[:, None], (4, L)).astype(jnp.float32)
    shape = (b, NW, n_sc, C)
    idx_s = scales.reshape(shape).astype(jnp.int32).transpose(1, 0, 2, 3)
    idx_r = rotations.reshape(shape).astype(jnp.int32).transpose(1, 0, 2, 3)
    idx_d = distances.reshape(shape).astype(jnp.int32).transpose(1, 0, 2, 3)

    def pack_bf16_rows(tbl):
        v = tbl.shape[0]
        xb = tbl.astype(jnp.bfloat16).reshape(v, d // (2 * L), 2, L)
        xb = xb.swapaxes(2, 3)
        return jax.lax.bitcast_convert_type(xb, jnp.int32).reshape(v, d // 2)

    out = _sc_fused_lookup(idx_s, idx_r, idx_d, pe.astype(jnp.float32),
                           pack_bf16_rows(scale_table),
                           pack_bf16_rows(rotation_table),
                           pack_bf16_rows(distance_table), wvec,
                           batch=b, seq_len=s, d=d)
    return out.reshape(b, s, d)


# speedup vs baseline: 1.6633x; 1.3801x over previous
import functools

import jax
import jax.numpy as jnp
from jax import lax
from jax.experimental import pallas as pl
from jax.experimental.pallas import tpu as pltpu
from jax.experimental.pallas import tpu_sc as plsc

NC = 2
NS = 16
L = 16
C = 16


@functools.partial(jax.jit, static_argnames=("batch", "seq_len", "d"))
def _sc_fused_lookup(idx_s, idx_r, idx_d, pe, st, rt, dt, wvec, *, batch, seq_len, d):
    NW = NC * NS
    s_per_w = seq_len // NW
    n_sc = s_per_w // C
    n_stage = n_sc * batch
    groups = d // L

    mesh = plsc.VectorSubcoreMesh(
        core_axis_name="c", subcore_axis_name="s",
        num_cores=NC, num_subcores=NS)

    @functools.partial(
        pl.kernel,
        out_type=jax.ShapeDtypeStruct((batch * seq_len, d), jnp.float32),
        mesh=mesh,
        scratch_types=[
            pltpu.VMEM((batch, n_sc, C), jnp.int32),
            pltpu.VMEM((batch, n_sc, C), jnp.int32),
            pltpu.VMEM((batch, n_sc, C), jnp.int32),
            pltpu.VMEM((C, d), jnp.float32),
            pltpu.VMEM((2, C, d), jnp.float32),
            pltpu.VMEM((2, C, d), jnp.float32),
            pltpu.VMEM((2, C, d), jnp.float32),
            pltpu.VMEM((4, L), jnp.float32),
            pltpu.SemaphoreType.DMA,
            pltpu.SemaphoreType.DMA,
            pltpu.SemaphoreType.DMA,
            pltpu.SemaphoreType.DMA,
            pltpu.SemaphoreType.DMA,
        ],
    )
    def body(sc_hbm, ro_hbm, di_hbm, pe_hbm, st_hbm, rt_hbm, dt_hbm, w_hbm,
             out_hbm, idx_sv, idx_rv, idx_dv, pe_v, g1, g2, g3, w_v,
             sem_g0, sem_g1, sem_o0, sem_o1, sem_pe):
        wid = lax.axis_index("s") * NC + lax.axis_index("c")
        s_base = wid * s_per_w

        for bb in range(batch):
            pltpu.sync_copy(sc_hbm.at[bb, wid], idx_sv.at[bb])
            pltpu.sync_copy(ro_hbm.at[bb, wid], idx_rv.at[bb])
            pltpu.sync_copy(di_hbm.at[bb, wid], idx_dv.at[bb])
        pltpu.sync_copy(w_hbm, w_v)
        w0 = w_v[0, :]
        w1 = w_v[1, :]
        w2 = w_v[2, :]
        w3 = w_v[3, :]

        sem_g = (sem_g0, sem_g1)
        sem_o = (sem_o0, sem_o1)
        gbufs = ((g1.at[0], g2.at[0], g3.at[0]), (g1.at[1], g2.at[1], g3.at[1]))

        def stage_tb(ls):
            return ls // batch, lax.rem(ls, batch)

        def issue_gathers23(ls, p):
            t, b = stage_tb(ls)
            _, b2, b3 = gbufs[p]
            pltpu.async_copy(rt_hbm.at[idx_rv.at[b, t]], b2, sem_g[p])
            pltpu.async_copy(dt_hbm.at[idx_dv.at[b, t]], b3, sem_g[p])

        def issue_gather1(ls, p):
            t, b = stage_tb(ls)
            pltpu.async_copy(st_hbm.at[idx_sv.at[b, t]], gbufs[p][0], sem_g[p])

        def wait_gathers(ls, p):
            t, b = stage_tb(ls)
            b1, b2, b3 = gbufs[p]
            pltpu.make_async_copy(st_hbm.at[idx_sv.at[b, t]], b1, sem_g[p]).wait()
            pltpu.make_async_copy(rt_hbm.at[idx_rv.at[b, t]], b2, sem_g[p]).wait()
            pltpu.make_async_copy(dt_hbm.at[idx_dv.at[b, t]], b3, sem_g[p]).wait()

        def out_rows(ls):
            t, b = stage_tb(ls)
            return b * seq_len + s_base + t * C

        def issue_out(ls, p):
            pltpu.async_copy(gbufs[p][0], out_hbm.at[pl.ds(out_rows(ls), C)],
                             sem_o[p])

        def wait_out(ls, p):
            pltpu.make_async_copy(gbufs[p][0],
                                  out_hbm.at[pl.ds(out_rows(ls), C)],
                                  sem_o[p]).wait()

        def issue_pe(t):
            pltpu.async_copy(pe_hbm.at[pl.ds(s_base + t * C, C)], pe_v, sem_pe)

        def wait_pe(t):
            pltpu.make_async_copy(pe_hbm.at[pl.ds(s_base + t * C, C)], pe_v,
                                  sem_pe).wait()

        def compute(p):
            b1, b2, b3 = gbufs[p]

            def row(i, carry2):
                def grp(jj, carry3):
                    for u in range(4):
                        sl = pl.ds((jj * 4 + u) * L, L)
                        b1[i, sl] = (pe_v[i, sl] * w0 + b1[i, sl] * w1
                                     + b2[i, sl] * w2 + b3[i, sl] * w3)
                    return carry3
                return lax.fori_loop(0, groups // 4, grp, carry2)

            lax.fori_loop(0, C, row, 0)

        issue_pe(0)
        issue_gathers23(0, 0)
        issue_gather1(0, 0)

        def iteration(k, carry):
            ls0 = 2 * k

            issue_gathers23(ls0 + 1, 1)

            @pl.when(k > 0)
            def _():
                wait_out(ls0 - 1, 1)
            issue_gather1(ls0 + 1, 1)

            @pl.when(lax.rem(ls0, batch) == 0)
            def _():
                wait_pe(ls0 // batch)
            wait_gathers(ls0, 0)
            compute(0)

            @pl.when((lax.rem(ls0, batch) == batch - 1)
                     & (ls0 // batch + 1 < n_sc))
            def _():
                issue_pe(ls0 // batch + 1)
            issue_out(ls0, 0)

            ls1 = ls0 + 1

            @pl.when(k < n_stage // 2 - 1)
            def _():
                issue_gathers23(ls1 + 1, 0)
            wait_out(ls1 - 1, 0)

            @pl.when(k < n_stage // 2 - 1)
            def _():
                issue_gather1(ls1 + 1, 0)

            @pl.when(lax.rem(ls1, batch) == 0)
            def _():
                wait_pe(ls1 // batch)
            wait_gathers(ls1, 1)
            compute(1)

            @pl.when((lax.rem(ls1, batch) == batch - 1)
                     & (ls1 // batch + 1 < n_sc))
            def _():
                issue_pe(ls1 // batch + 1)
            issue_out(ls1, 1)
            return carry

        lax.fori_loop(0, n_stage // 2, iteration, 0)
        wait_out(n_stage - 1, 1)

    return body(idx_s, idx_r, idx_d, pe, st, rt, dt, wvec)


def kernel(positions, scales, rotations, distances, pe, scale_table,
           rotation_table, distance_table, fusion_weights):
    b, s = positions.shape
    d = pe.shape[1]
    NW = NC * NS
    n_sc = s // NW // C
    w = jax.nn.softmax(fusion_weights.astype(jnp.float32), axis=0)
    wvec = jnp.broadcast_to(w[:, None], (4, L)).astype(jnp.float32)
    shape = (b, NW, n_sc, C)
    idx_s = scales.reshape(shape).astype(jnp.int32)
    idx_r = rotations.reshape(shape).astype(jnp.int32)
    idx_d = distances.reshape(shape).astype(jnp.int32)
    out = _sc_fused_lookup(idx_s, idx_r, idx_d, pe, scale_table,
                           rotation_table, distance_table, wvec,
                           batch=b, seq_len=s, d=d)
    return out.reshape(b, s, d)


# speedup vs baseline: 1.7166x; 1.0320x over previous
import functools

import jax
import jax.numpy as jnp
from jax import lax
from jax.experimental import pallas as pl
from jax.experimental.pallas import tpu as pltpu
from jax.experimental.pallas import tpu_sc as plsc

NC = 2
NS = 16
L = 16
C = 16


@functools.partial(jax.jit, static_argnames=("batch", "seq_len", "d"))
def _sc_fused_lookup(idx_s, idx_r, idx_d, pe, st, rt, dt, wvec, *, batch, seq_len, d):
    NW = NC * NS
    s_per_w = seq_len // NW
    n_sc = s_per_w // C
    n_stage = n_sc * batch
    groups = d // L

    mesh = plsc.VectorSubcoreMesh(
        core_axis_name="c", subcore_axis_name="s",
        num_cores=NC, num_subcores=NS)

    @functools.partial(
        pl.kernel,
        out_type=jax.ShapeDtypeStruct((batch * seq_len, d), jnp.float32),
        mesh=mesh,
        scratch_types=[
            pltpu.VMEM((batch, n_sc, C), jnp.int32),
            pltpu.VMEM((batch, n_sc, C), jnp.int32),
            pltpu.VMEM((batch, n_sc, C), jnp.int32),
            pltpu.VMEM((C, d), jnp.float32),
            pltpu.VMEM((2, C, d), jnp.float32),
            pltpu.VMEM((2, C, d), jnp.float32),
            pltpu.VMEM((2, C, d), jnp.float32),
            pltpu.VMEM((4, L), jnp.float32),
            pltpu.SemaphoreType.DMA,
            pltpu.SemaphoreType.DMA,
            pltpu.SemaphoreType.DMA,
            pltpu.SemaphoreType.DMA,
            pltpu.SemaphoreType.DMA,
        ],
    )
    def body(sc_hbm, ro_hbm, di_hbm, pe_hbm, st_hbm, rt_hbm, dt_hbm, w_hbm,
             out_hbm, idx_sv, idx_rv, idx_dv, pe_v, g1, g2, g3, w_v,
             sem_g0, sem_g1, sem_o0, sem_o1, sem_pe):
        wid = lax.axis_index("s") * NC + lax.axis_index("c")
        s_base = wid * s_per_w

        pltpu.sync_copy(sc_hbm.at[wid], idx_sv)
        pltpu.sync_copy(ro_hbm.at[wid], idx_rv)
        pltpu.sync_copy(di_hbm.at[wid], idx_dv)
        pltpu.sync_copy(w_hbm, w_v)
        w0 = w_v[0, :]
        w1 = w_v[1, :]
        w2 = w_v[2, :]
        w3 = w_v[3, :]

        sem_g = (sem_g0, sem_g1)
        sem_o = (sem_o0, sem_o1)
        gbufs = ((g1.at[0], g2.at[0], g3.at[0]), (g1.at[1], g2.at[1], g3.at[1]))

        def stage_tb(ls):
            return ls // batch, lax.rem(ls, batch)

        def issue_gathers23(ls, p):
            t, b = stage_tb(ls)
            _, b2, b3 = gbufs[p]
            pltpu.async_copy(rt_hbm.at[idx_rv.at[b, t]], b2, sem_g[p])
            pltpu.async_copy(dt_hbm.at[idx_dv.at[b, t]], b3, sem_g[p])

        def issue_gather1(ls, p):
            t, b = stage_tb(ls)
            pltpu.async_copy(st_hbm.at[idx_sv.at[b, t]], gbufs[p][0], sem_g[p])

        def wait_gathers(ls, p):
            t, b = stage_tb(ls)
            b1, b2, b3 = gbufs[p]
            pltpu.make_async_copy(st_hbm.at[idx_sv.at[b, t]], b1, sem_g[p]).wait()
            pltpu.make_async_copy(rt_hbm.at[idx_rv.at[b, t]], b2, sem_g[p]).wait()
            pltpu.make_async_copy(dt_hbm.at[idx_dv.at[b, t]], b3, sem_g[p]).wait()

        def out_rows(ls):
            t, b = stage_tb(ls)
            return b * seq_len + s_base + t * C

        def issue_out(ls, p):
            pltpu.async_copy(gbufs[p][0], out_hbm.at[pl.ds(out_rows(ls), C)],
                             sem_o[p])

        def wait_out(ls, p):
            pltpu.make_async_copy(gbufs[p][0],
                                  out_hbm.at[pl.ds(out_rows(ls), C)],
                                  sem_o[p]).wait()

        def issue_pe(t):
            pltpu.async_copy(pe_hbm.at[pl.ds(s_base + t * C, C)], pe_v, sem_pe)

        def wait_pe(t):
            pltpu.make_async_copy(pe_hbm.at[pl.ds(s_base + t * C, C)], pe_v,
                                  sem_pe).wait()

        def compute(p):
            b1, b2, b3 = gbufs[p]

            def row(i, carry2):
                def grp(jj, carry3):
                    for u in range(4):
                        sl = pl.ds((jj * 4 + u) * L, L)
                        b1[i, sl] = (pe_v[i, sl] * w0 + b1[i, sl] * w1
                                     + b2[i, sl] * w2 + b3[i, sl] * w3)
                    return carry3
                return lax.fori_loop(0, groups // 4, grp, carry2)

            lax.fori_loop(0, C, row, 0)

        issue_pe(0)
        issue_gathers23(0, 0)
        issue_gather1(0, 0)

        def iteration(k, carry):
            ls0 = 2 * k

            issue_gathers23(ls0 + 1, 1)

            @pl.when(k > 0)
            def _():
                wait_out(ls0 - 1, 1)
            issue_gather1(ls0 + 1, 1)

            @pl.when(lax.rem(ls0, batch) == 0)
            def _():
                wait_pe(ls0 // batch)
            wait_gathers(ls0, 0)
            compute(0)

            @pl.when((lax.rem(ls0, batch) == batch - 1)
                     & (ls0 // batch + 1 < n_sc))
            def _():
                issue_pe(ls0 // batch + 1)
            issue_out(ls0, 0)

            ls1 = ls0 + 1

            @pl.when(k < n_stage // 2 - 1)
            def _():
                issue_gathers23(ls1 + 1, 0)
            wait_out(ls1 - 1, 0)

            @pl.when(k < n_stage // 2 - 1)
            def _():
                issue_gather1(ls1 + 1, 0)

            @pl.when(lax.rem(ls1, batch) == 0)
            def _():
                wait_pe(ls1 // batch)
            wait_gathers(ls1, 1)
            compute(1)

            @pl.when((lax.rem(ls1, batch) == batch - 1)
                     & (ls1 // batch + 1 < n_sc))
            def _():
                issue_pe(ls1 // batch + 1)
            issue_out(ls1, 1)
            return carry

        lax.fori_loop(0, n_stage // 2, iteration, 0)
        wait_out(n_stage - 1, 1)

    return body(idx_s, idx_r, idx_d, pe, st, rt, dt, wvec)


def kernel(positions, scales, rotations, distances, pe, scale_table,
           rotation_table, distance_table, fusion_weights):
    b, s = positions.shape
    d = pe.shape[1]
    NW = NC * NS
    n_sc = s // NW // C
    w = jax.nn.softmax(fusion_weights.astype(jnp.float32), axis=0)
    wvec = jnp.broadcast_to(w[:, None], (4, L)).astype(jnp.float32)
    shape = (b, NW, n_sc, C)
    idx_s = scales.reshape(shape).astype(jnp.int32).transpose(1, 0, 2, 3)
    idx_r = rotations.reshape(shape).astype(jnp.int32).transpose(1, 0, 2, 3)
    idx_d = distances.reshape(shape).astype(jnp.int32).transpose(1, 0, 2, 3)
    out = _sc_fused_lookup(idx_s, idx_r, idx_d, pe, scale_table,
                           rotation_table, distance_table, wvec,
                           batch=b, seq_len=s, d=d)
    return out.reshape(b, s, d)
